# baseline (speedup 1.0000x reference)
import jax
import jax.numpy as jnp
from jax import lax
from jax.experimental import pallas as pl
from jax.experimental.pallas import tpu as pltpu

_EPS = 1e-8
_TILE_B = 8192
_GRAM_BLK = 512


def _normalize_kernel(t_ref, o_ref):
    t = t_ref[...]
    nrm = jnp.sqrt(jnp.sum(t * t, axis=1, keepdims=True))
    o_ref[...] = t * (1.0 / jnp.maximum(nrm, _EPS))


def _gram_kernel(u_ref, m_ref, o_ref):
    c = lax.dot_general(u_ref[...], m_ref[...], (((1,), (1,)), ((), ())),
                        preferred_element_type=jnp.float32)
    o_ref[...] = c.astype(jnp.bfloat16)


def _pair_gather_kernel(row_ref, mlow_ref, c_ref, o_ref, slab_t):
    tb = o_ref.shape[2]
    for mi in range(tb):
        slab_t[mi, 0] = c_ref[row_ref[0, 0, mi], 0]
    tile = slab_t[...].reshape(tb, 128)
    ones11 = jnp.ones((1, 1), jnp.float32)
    mrow = mlow_ref[0].astype(jnp.float32)
    mcol_f = lax.dot_general(mrow, ones11, (((0,), (0,)), ((), ())),
                             preferred_element_type=jnp.float32)
    mlow = jnp.round(mcol_f).astype(jnp.int32)
    lane = mlow >> 1
    vals = jnp.take_along_axis(tile, lane, axis=1)
    shifted = jnp.where(mlow & 1 == 0, vals << 16, vals)
    bits = shifted & jnp.int32(-65536)
    valf = pltpu.bitcast(bits, jnp.float32)
    o_ref[0] = lax.dot_general(ones11, valf, (((0,), (1,)), ((), ())),
                               preferred_element_type=jnp.float32)


def kernel(user_table, movie_table, user_id, movie_id):
    B = int(user_id.shape[0])
    Nu, D = user_table.shape
    Nm = movie_table.shape[0]

    both = jnp.concatenate([user_table, movie_table], axis=0)
    nrows = Nu + Nm
    nblk = min(_GRAM_BLK, nrows)
    while nrows % nblk:
        nblk //= 2
    normed = pl.pallas_call(
        _normalize_kernel,
        out_shape=jax.ShapeDtypeStruct((nrows, D), jnp.float32),
        grid=(nrows // nblk,),
        in_specs=[pl.BlockSpec((nblk, D), lambda i: (i, 0))],
        out_specs=pl.BlockSpec((nblk, D), lambda i: (i, 0)),
        compiler_params=pltpu.CompilerParams(
            dimension_semantics=("arbitrary",)),
    )(both)
    u_hat = normed[:Nu]
    m_hat = normed[Nu:]

    gblk = min(_GRAM_BLK, Nu)
    while Nu % gblk:
        gblk //= 2
    gram = pl.pallas_call(
        _gram_kernel,
        out_shape=jax.ShapeDtypeStruct((Nu, Nm), jnp.bfloat16),
        grid=(Nu // gblk,),
        in_specs=[pl.BlockSpec((gblk, D), lambda i: (i, 0)),
                  pl.BlockSpec((Nm, D), lambda i: (0, 0))],
        out_specs=pl.BlockSpec((gblk, Nm), lambda i: (i, 0)),
        compiler_params=pltpu.CompilerParams(
            dimension_semantics=("arbitrary",)),
        cost_estimate=pl.CostEstimate(
            flops=int(2 * Nu * Nm * D), transcendentals=0,
            bytes_accessed=int((Nu + Nm) * D * 4 + Nu * Nm * 2)),
    )(u_hat, m_hat)

    c32 = lax.bitcast_convert_type(
        gram.reshape(Nu, Nm // 2, 2), jnp.int32)
    c32 = c32.reshape(Nu * (Nm // 256), 1, 128)

    tile_b = min(_TILE_B, max(128, B))
    num_tiles = pl.cdiv(B, tile_b)
    Bp = num_tiles * tile_b

    uid = user_id.astype(jnp.int32)
    mid = movie_id.astype(jnp.int32)
    pad = Bp - B
    if pad:
        uid = jnp.concatenate([uid, jnp.zeros((pad,), jnp.int32)])
        mid = jnp.concatenate([mid, jnp.zeros((pad,), jnp.int32)])
    row = uid * (Nm // 256) + (mid >> 8)
    mlow = mid & 255
    row3 = row.reshape(num_tiles, 1, tile_b)
    mlow3 = mlow.reshape(num_tiles, 1, tile_b)

    gram_bytes = Nu * Nm * 2
    cost = pl.CostEstimate(
        flops=int(10 * Bp),
        transcendentals=0,
        bytes_accessed=int(gram_bytes + 3 * Bp * 4),
    )
    vmem_limit = int(gram_bytes + 6 * tile_b * 128 * 4 + (8 << 20))

    def _c_spec(mode):
        if mode is None:
            return pl.BlockSpec((Nu * (Nm // 256), 1, 128),
                                lambda i: (0, 0, 0))
        return pl.BlockSpec((Nu * (Nm // 256), 1, 128), lambda i: (0, 0, 0),
                            pipeline_mode=mode)

    try:
        c_mode = pl.Buffered(1)
    except Exception:
        c_mode = None

    out = pl.pallas_call(
        _pair_gather_kernel,
        out_shape=jax.ShapeDtypeStruct((num_tiles, 1, tile_b), jnp.float32),
        grid=(num_tiles,),
        in_specs=[
            pl.BlockSpec((1, 1, tile_b), lambda i: (i, 0, 0),
                         memory_space=pltpu.SMEM),
            pl.BlockSpec((1, 1, tile_b), lambda i: (i, 0, 0)),
            _c_spec(c_mode),
        ],
        out_specs=pl.BlockSpec((1, 1, tile_b), lambda i: (i, 0, 0)),
        scratch_shapes=[
            pltpu.VMEM((tile_b, 1, 128), jnp.int32),
        ],
        compiler_params=pltpu.CompilerParams(
            dimension_semantics=("arbitrary",),
            vmem_limit_bytes=vmem_limit,
        ),
        cost_estimate=cost,
    )(row3, mlow3, c32)
    return out.reshape(-1)[:B]


# speedup vs baseline: 1.0016x; 1.0016x over previous
import jax
import jax.numpy as jnp
from jax import lax
from jax.experimental import pallas as pl
from jax.experimental.pallas import tpu as pltpu

_EPS = 1e-8
_TILE_B = 8192
_GRAM_BLK = 512


def _normalize_kernel(t_ref, o_ref):
    t = t_ref[...]
    nrm = jnp.sqrt(jnp.sum(t * t, axis=1, keepdims=True))
    o_ref[...] = t * (1.0 / jnp.maximum(nrm, _EPS))


def _gram_kernel(u_ref, m_ref, o_ref):
    c = lax.dot_general(u_ref[...], m_ref[...], (((1,), (1,)), ((), ())),
                        preferred_element_type=jnp.float32)
    o_ref[...] = c.astype(jnp.bfloat16)


def _pair_gather_kernel(row_ref, mlow_ref, c_ref, o_ref, slab_a, slab_b):
    tb = o_ref.shape[2]
    hb = tb // 2
    for mi in range(hb):
        slab_a[mi, 0] = c_ref[row_ref[0, 0, mi], 0]
        slab_b[mi, 0] = c_ref[row_ref[0, 0, hb + mi], 0]
    tile = jnp.concatenate(
        [slab_a[...].reshape(hb, 128), slab_b[...].reshape(hb, 128)],
        axis=0)
    ones11 = jnp.ones((1, 1), jnp.float32)
    mrow = mlow_ref[0].astype(jnp.float32)
    mcol_f = lax.dot_general(mrow, ones11, (((0,), (0,)), ((), ())),
                             preferred_element_type=jnp.float32)
    mlow = jnp.round(mcol_f).astype(jnp.int32)
    lane = mlow >> 1
    vals = jnp.take_along_axis(tile, lane, axis=1)
    shifted = jnp.where(mlow & 1 == 0, vals << 16, vals)
    bits = shifted & jnp.int32(-65536)
    valf = pltpu.bitcast(bits, jnp.float32)
    o_ref[0] = lax.dot_general(ones11, valf, (((0,), (1,)), ((), ())),
                               preferred_element_type=jnp.float32)


def kernel(user_table, movie_table, user_id, movie_id):
    B = int(user_id.shape[0])
    Nu, D = user_table.shape
    Nm = movie_table.shape[0]

    both = jnp.concatenate([user_table, movie_table], axis=0)
    nrows = Nu + Nm
    nblk = min(_GRAM_BLK, nrows)
    while nrows % nblk:
        nblk //= 2
    normed = pl.pallas_call(
        _normalize_kernel,
        out_shape=jax.ShapeDtypeStruct((nrows, D), jnp.float32),
        grid=(nrows // nblk,),
        in_specs=[pl.BlockSpec((nblk, D), lambda i: (i, 0))],
        out_specs=pl.BlockSpec((nblk, D), lambda i: (i, 0)),
        compiler_params=pltpu.CompilerParams(
            dimension_semantics=("arbitrary",)),
    )(both)
    u_hat = normed[:Nu]
    m_hat = normed[Nu:]

    gblk = min(_GRAM_BLK, Nu)
    while Nu % gblk:
        gblk //= 2
    gram = pl.pallas_call(
        _gram_kernel,
        out_shape=jax.ShapeDtypeStruct((Nu, Nm), jnp.bfloat16),
        grid=(Nu // gblk,),
        in_specs=[pl.BlockSpec((gblk, D), lambda i: (i, 0)),
                  pl.BlockSpec((Nm, D), lambda i: (0, 0))],
        out_specs=pl.BlockSpec((gblk, Nm), lambda i: (i, 0)),
        compiler_params=pltpu.CompilerParams(
            dimension_semantics=("arbitrary",)),
        cost_estimate=pl.CostEstimate(
            flops=int(2 * Nu * Nm * D), transcendentals=0,
            bytes_accessed=int((Nu + Nm) * D * 4 + Nu * Nm * 2)),
    )(u_hat, m_hat)

    c32 = lax.bitcast_convert_type(
        gram.reshape(Nu, Nm // 2, 2), jnp.int32)
    c32 = c32.reshape(Nu * (Nm // 256), 1, 128)

    tile_b = min(_TILE_B, max(128, B))
    num_tiles = pl.cdiv(B, tile_b)
    Bp = num_tiles * tile_b

    uid = user_id.astype(jnp.int32)
    mid = movie_id.astype(jnp.int32)
    pad = Bp - B
    if pad:
        uid = jnp.concatenate([uid, jnp.zeros((pad,), jnp.int32)])
        mid = jnp.concatenate([mid, jnp.zeros((pad,), jnp.int32)])
    row = uid * (Nm // 256) + (mid >> 8)
    mlow = mid & 255
    row3 = row.reshape(num_tiles, 1, tile_b)
    mlow3 = mlow.reshape(num_tiles, 1, tile_b)

    gram_bytes = Nu * Nm * 2
    cost = pl.CostEstimate(
        flops=int(10 * Bp),
        transcendentals=0,
        bytes_accessed=int(gram_bytes + 3 * Bp * 4),
    )
    vmem_limit = int(gram_bytes + 6 * tile_b * 128 * 4 + (8 << 20))

    def _c_spec(mode):
        if mode is None:
            return pl.BlockSpec((Nu * (Nm // 256), 1, 128),
                                lambda i: (0, 0, 0))
        return pl.BlockSpec((Nu * (Nm // 256), 1, 128), lambda i: (0, 0, 0),
                            pipeline_mode=mode)

    try:
        c_mode = pl.Buffered(1)
    except Exception:
        c_mode = None

    out = pl.pallas_call(
        _pair_gather_kernel,
        out_shape=jax.ShapeDtypeStruct((num_tiles, 1, tile_b), jnp.float32),
        grid=(num_tiles,),
        in_specs=[
            pl.BlockSpec((1, 1, tile_b), lambda i: (i, 0, 0),
                         memory_space=pltpu.SMEM),
            pl.BlockSpec((1, 1, tile_b), lambda i: (i, 0, 0)),
            _c_spec(c_mode),
        ],
        out_specs=pl.BlockSpec((1, 1, tile_b), lambda i: (i, 0, 0)),
        scratch_shapes=[
            pltpu.VMEM((tile_b // 2, 1, 128), jnp.int32),
            pltpu.VMEM((tile_b // 2, 1, 128), jnp.int32),
        ],
        compiler_params=pltpu.CompilerParams(
            dimension_semantics=("arbitrary",),
            vmem_limit_bytes=vmem_limit,
        ),
        cost_estimate=cost,
    )(row3, mlow3, c32)
    return out.reshape(-1)[:B]
